# baseline (speedup 1.0000x reference)
import functools

import jax
import jax.numpy as jnp
from jax import lax
from jax.experimental import pallas as pl
from jax.experimental.pallas import tpu as pltpu
from jax.experimental.pallas import tpu_sc as plsc

NC, NS = 2, 16
NW = NC * NS
D = 32
IDX_BLK = 128
BLKS_PER_CHUNK = 8
CHUNK = IDX_BLK * BLKS_PER_CHUNK
NBUF = 2


def _sc_gather(table, idx2d):
    n = idx2d.shape[0] * IDX_BLK
    per_w = n // NW
    n_chunks = per_w // CHUNK

    mesh = plsc.VectorSubcoreMesh(
        core_axis_name="c", subcore_axis_name="s",
        num_cores=NC, num_subcores=NS)

    @functools.partial(
        pl.kernel,
        out_type=jax.ShapeDtypeStruct((n, D), jnp.float32),
        mesh=mesh,
        scratch_types=[
            pltpu.VMEM((NBUF, BLKS_PER_CHUNK, IDX_BLK), jnp.int32),
            pltpu.VMEM((NBUF, CHUNK, D), jnp.float32),
            pltpu.SemaphoreType.DMA((NBUF,)),
            pltpu.SemaphoreType.DMA((NBUF,)),
            pltpu.SemaphoreType.DMA,
        ],
        compiler_params=pltpu.CompilerParams(use_tc_tiling_on_sc=False),
    )
    def k(table_hbm, idx_hbm, out_hbm, idx_v, rows_v, sem_idx, sem_out, sem_g):
        wid = lax.axis_index("s") * NC + lax.axis_index("c")
        chunk0 = wid * n_chunks

        def idx_copy(c, b):
            return pltpu.make_async_copy(
                idx_hbm.at[pl.ds((chunk0 + c) * BLKS_PER_CHUNK, BLKS_PER_CHUNK), :],
                idx_v.at[b], sem_idx.at[b])

        def out_copy(c, b):
            return pltpu.make_async_copy(
                rows_v.at[b], out_hbm.at[pl.ds((chunk0 + c) * CHUNK, CHUNK), :],
                sem_out.at[b])

        for b in range(NBUF):
            idx_copy(b, b).start()

        def step2(i2, carry):
            for b in range(NBUF):
                c = i2 * NBUF + b
                idx_copy(c, b).wait()
                @pl.when(c >= NBUF)
                def _():
                    out_copy(c - NBUF, b).wait()
                gathers = [
                    pltpu.async_copy(
                        table_hbm.at[idx_v.at[b, j]],
                        rows_v.at[b, pl.ds(j * IDX_BLK, IDX_BLK), :],
                        sem_g)
                    for j in range(BLKS_PER_CHUNK)
                ]
                for g in gathers:
                    g.wait()
                out_copy(c, b).start()
                @pl.when(c + NBUF < n_chunks)
                def _():
                    idx_copy(c + NBUF, b).start()
            return carry

        lax.fori_loop(0, n_chunks // NBUF, step2, 0)
        for b in range(NBUF):
            out_copy(n_chunks - NBUF + b, b).wait()

    return k(table, idx2d)


def _tc_concat_rows(wtT, wfT):
    n_t, n_f = wtT.shape[1], wfT.shape[1]
    A, RR = 8, 1250
    R = A * RR
    NT, NF = n_t // R, n_f // R

    wtT4 = wtT.reshape(D, NT, A, RR)
    wfT4 = wfT.reshape(D, NF, A, RR)

    def body(wt_ref, wf_ref, out_ref):
        p = pl.program_id(0)

        def emit(ref):
            x = ref[:, 0, :, :]
            for a in range(A):
                out_ref[0, a] = x[:, a, :].T

        @pl.when(p < NT)
        def _():
            emit(wt_ref)

        @pl.when(p >= NT)
        def _():
            emit(wf_ref)

    out4 = pl.pallas_call(
        body,
        grid=(NT + NF,),
        in_specs=[
            pl.BlockSpec((D, 1, A, RR), lambda p: (0, jnp.minimum(p, NT - 1), 0, 0)),
            pl.BlockSpec((D, 1, A, RR), lambda p: (0, jnp.maximum(p - NT, 0), 0, 0)),
        ],
        out_specs=pl.BlockSpec((1, A, RR, D), lambda p: (p, 0, 0, 0)),
        out_shape=jax.ShapeDtypeStruct((NT + NF, A, RR, D), jnp.float32),
    )(wtT4, wfT4)
    return out4.reshape((n_t + n_f), D)


def kernel(idx, W_train, W_frozen):
    table = _tc_concat_rows(W_train.T, W_frozen.T)
    idx2d = idx.reshape(-1, IDX_BLK)
    out = _sc_gather(table, idx2d)
    return out.reshape(idx.shape + (D,))
